# Initial kernel scaffold; baseline (speedup 1.0000x reference)
import jax
import jax.numpy as jnp
from jax.experimental import pallas as pl


def kernel(pred, gt, mask):
    raise NotImplementedError("write your pallas kernel here")


# speedup vs baseline: 17.6122x; 17.6122x over previous
import jax
import jax.numpy as jnp
from jax.experimental import pallas as pl
from jax.experimental.pallas import tpu as pltpu

_B, _H, _W = 8, 512, 512
_R, _C = 2048, 1024
_BR = 256
_NEG_RATIO = 3
_EPS = 1e-06
_MAX_BITS = 0x42C80000


def _stage1(pred_ref, gt_ref, mask_ref, vbits_ref, psum_ref, pcnt_ref,
            ncnt_ref):
    i = pl.program_id(0)
    p = pred_ref[...]
    g = gt_ref[...]
    m = mask_ref[...]
    log_p = jnp.maximum(jnp.log(p), -100.0)
    log_1mp = jnp.maximum(jnp.log(1.0 - p), -100.0)
    loss = -(g * log_p + (1.0 - g) * log_1mp)
    pos = g * m
    neg = (1.0 - g) * m
    v = neg * loss
    vbits_ref[...] = jax.lax.bitcast_convert_type(v, jnp.int32)

    @pl.when(i == 0)
    def _init():
        psum_ref[...] = jnp.zeros((1, 1), jnp.float32)
        pcnt_ref[...] = jnp.zeros((1, 1), jnp.float32)
        ncnt_ref[...] = jnp.zeros((1, 1), jnp.float32)

    psum_ref[...] += jnp.sum(pos * loss, keepdims=True)
    pcnt_ref[...] += jnp.sum(pos, keepdims=True)
    ncnt_ref[...] += jnp.sum(neg, keepdims=True)


def _stage2(k_ref, vbits_ref, tbits_ref, cntgt_ref, sumgt_ref):
    k = k_ref[0]
    bits = vbits_ref[...]

    def body(_, carry):
        lo, hi = carry
        mid = lo + (hi - lo + 1) // 2
        cnt = jnp.sum((bits >= mid).astype(jnp.int32))
        ge = cnt >= k
        return (jnp.where(ge, mid, lo), jnp.where(ge, hi, mid - 1))

    lo, _ = jax.lax.fori_loop(
        0, 31, body, (jnp.int32(0), jnp.int32(_MAX_BITS)))

    vals = jax.lax.bitcast_convert_type(bits, jnp.float32)
    gt_mask = bits > lo
    tbits_ref[...] = jnp.reshape(lo, (1, 1))
    cntgt_ref[...] = jnp.sum(gt_mask.astype(jnp.int32), keepdims=True)
    sumgt_ref[...] = jnp.sum(jnp.where(gt_mask, vals, 0.0), keepdims=True)


def kernel(pred, gt, mask):
    p2 = pred.reshape(_R, _C)
    g2 = gt.reshape(_R, _C)
    m2 = mask.reshape(_R, _C)

    vbits, psum, pcnt, ncnt = pl.pallas_call(
        _stage1,
        grid=(_R // _BR,),
        in_specs=[pl.BlockSpec((_BR, _C), lambda i: (i, 0))] * 3,
        out_specs=[
            pl.BlockSpec((_BR, _C), lambda i: (i, 0)),
            pl.BlockSpec((1, 1), lambda i: (0, 0)),
            pl.BlockSpec((1, 1), lambda i: (0, 0)),
            pl.BlockSpec((1, 1), lambda i: (0, 0)),
        ],
        out_shape=[
            jax.ShapeDtypeStruct((_R, _C), jnp.int32),
            jax.ShapeDtypeStruct((1, 1), jnp.float32),
            jax.ShapeDtypeStruct((1, 1), jnp.float32),
            jax.ShapeDtypeStruct((1, 1), jnp.float32),
        ],
    )(p2, g2, m2)

    pos_cnt = pcnt[0, 0].astype(jnp.int32)
    neg_cnt = jnp.minimum(
        ncnt[0, 0], (pos_cnt * _NEG_RATIO).astype(jnp.float32)
    ).astype(jnp.int32)

    tbits, cntgt, sumgt = pl.pallas_call(
        _stage2,
        in_specs=[
            pl.BlockSpec(memory_space=pltpu.SMEM),
            pl.BlockSpec((_R, _C), lambda: (0, 0)),
        ],
        out_specs=[
            pl.BlockSpec((1, 1), lambda: (0, 0)),
            pl.BlockSpec((1, 1), lambda: (0, 0)),
            pl.BlockSpec((1, 1), lambda: (0, 0)),
        ],
        out_shape=[
            jax.ShapeDtypeStruct((1, 1), jnp.int32),
            jax.ShapeDtypeStruct((1, 1), jnp.int32),
            jax.ShapeDtypeStruct((1, 1), jnp.float32),
        ],
    )(neg_cnt.reshape(1), vbits)

    t = jax.lax.bitcast_convert_type(tbits[0, 0], jnp.float32)
    neg_top = jnp.where(
        neg_cnt > 0,
        sumgt[0, 0] + (neg_cnt - cntgt[0, 0]).astype(jnp.float32) * t,
        0.0,
    )
    denom = (pos_cnt + neg_cnt).astype(jnp.float32) + _EPS
    return (psum[0, 0] + neg_top) / denom
